# baseline (speedup 1.0000x reference)
import jax
import jax.numpy as jnp
from jax import lax
from jax.experimental import pallas as pl
from jax.experimental.pallas import tpu as pltpu
from jax.experimental.pallas import tpu_sc as plsc

N_SUB = 16
N_CORE = 2
CHUNK_T = 16
CHUNK_I = 128
CHUNK = CHUNK_T * CHUNK_I
NW = N_CORE * N_SUB


def _sc_prop(n, e, w, gather):
  nchunks = e // CHUNK
  npr = n // N_SUB
  nz = npr // 2

  mesh = plsc.VectorSubcoreMesh(core_axis_name="c", subcore_axis_name="s",
                                num_cores=N_CORE, num_subcores=N_SUB)

  scratch = [
      pltpu.VMEM((2, CHUNK_T, CHUNK_I), jnp.int32),
      pltpu.VMEM((3, CHUNK_T, CHUNK_I), jnp.int32),
      pltpu.VMEM((2, CHUNK, w), jnp.float32),
      pltpu.VMEM_SHARED((n, w), jnp.float32),
      pltpu.SemaphoreType.DMA((2,)),
      pltpu.SemaphoreType.DMA((3,)),
      pltpu.SemaphoreType.DMA,
      pltpu.SemaphoreType.DMA((2,)),
  ]

  def body(src_hbm, row_hbm, col_hbm, zero_hbm, out_hbm, idxr, idxc, rows,
           *rest):
    acc_sp, irsem, icsem, gsem, ssem = rest
    c = lax.axis_index("c")
    s = lax.axis_index("s")
    wid = c * N_SUB + s
    base = s * npr

    if not gather:
      pltpu.sync_copy(src_hbm.at[pl.ds(0, CHUNK)], rows.at[0])
    for z in range(2):
      pltpu.sync_copy(zero_hbm, acc_sp.at[pl.ds(base + z * nz, nz)])
    plsc.subcore_barrier()

    ntrips = (nchunks - wid + NW - 1) // NW

    def start_idx(i):
      cid = wid + i * NW
      if gather:
        pltpu.async_copy(row_hbm.at[cid], idxr.at[i % 2], irsem.at[i % 2])
      pltpu.async_copy(col_hbm.at[cid], idxc.at[i % 3], icsem.at[i % 3])

    def wait_scatters(p):
      for j in range(CHUNK_T):
        pltpu.make_async_copy(rows.at[0, pl.ds(j * CHUNK_I, CHUNK_I)],
                              acc_sp.at[idxc.at[0, 0]], ssem.at[p]).wait()

    start_idx(0)

    def trip(i, carry):
      b2 = i % 2
      b3 = i % 3

      @pl.when(i >= 2)
      def _():
        wait_scatters(b2)

      @pl.when(i + 1 < ntrips)
      def _():
        start_idx(i + 1)

      if gather:
        pltpu.make_async_copy(row_hbm.at[0], idxr.at[b2],
                              irsem.at[b2]).wait()
      pltpu.make_async_copy(col_hbm.at[0], idxc.at[b3], icsem.at[b3]).wait()

      if gather:
        gcps = [
            pltpu.async_copy(src_hbm.at[idxr.at[b2, j]],
                             rows.at[b2, pl.ds(j * CHUNK_I, CHUNK_I)], gsem)
            for j in range(CHUNK_T)
        ]
        for cp in gcps:
          cp.wait()
      for j in range(CHUNK_T):
        pltpu.async_copy(
            rows.at[b2 if gather else 0, pl.ds(j * CHUNK_I, CHUNK_I)],
            acc_sp.at[idxc.at[b3, j]], ssem.at[b2], add=True)
      return carry

    lax.fori_loop(0, ntrips, trip, 0)
    wait_scatters(0)
    wait_scatters(1)
    plsc.subcore_barrier()

    pltpu.sync_copy(acc_sp.at[pl.ds(base, npr)],
                    out_hbm.at[c, pl.ds(base, npr)])

  return pl.kernel(
      body,
      out_type=jax.ShapeDtypeStruct((N_CORE, n, w), jnp.float32),
      mesh=mesh,
      scratch_types=scratch,
      compiler_params=pltpu.CompilerParams(use_tc_tiling_on_sc=False),
  )


_GRID = 8


def _tc_call(fn, v, in_kinds, n_out):
  rv = v // _GRID

  def spec(kd):
    if isinstance(kd, tuple) and kd[0] == 'F':
      shape = kd[1]
      return pl.BlockSpec(shape, lambda i: (0,) * len(shape))
    if kd == 2:
      return pl.BlockSpec((2, rv, 128), lambda i: (0, i, 0))
    return pl.BlockSpec((rv, 128), lambda i: (i, 0))

  return pl.pallas_call(
      fn,
      grid=(_GRID,),
      in_specs=[spec(kd) for kd in in_kinds],
      out_specs=[spec(1) for _ in range(n_out)],
      out_shape=[jax.ShapeDtypeStruct((v, 128), jnp.float32)] * n_out,
  )


def _stage_a(degp_ref, x_ref, d8_ref, p0_ref):
  deg = degp_ref[0] + degp_ref[1]
  d8 = jnp.where(deg > 0, lax.rsqrt(jnp.maximum(deg, 1e-12)), 0.0)
  d8_ref[...] = d8
  p0_ref[...] = x_ref[...] * d8


def _stage_pk(qp_ref, d8_ref, pk_ref):
  d8 = d8_ref[...]
  pk_ref[...] = (qp_ref[0] + qp_ref[1]) * d8 * d8


def _stage_mid(x_ref, q1_ref, q2_ref, q3_ref, d8_ref, bd1_ref, b1t_ref,
               bd2_ref, p3_ref, c_ref, s3_ref):
  d8 = d8_ref[...]
  g1 = (q1_ref[0] + q1_ref[1]) * d8
  g2 = (q2_ref[0] + q2_ref[1]) * d8
  g3 = (q3_ref[0] + q3_ref[1]) * d8
  o = jnp.dot(x_ref[...], bd1_ref[0], preferred_element_type=jnp.float32)
  o += jnp.dot(g1, bd1_ref[1], preferred_element_type=jnp.float32)
  o += jnp.dot(g2, bd1_ref[2], preferred_element_type=jnp.float32)
  o += jnp.dot(g3, bd1_ref[3], preferred_element_type=jnp.float32)
  h = jnp.maximum(o + b1t_ref[...][None, :], 0.0)
  cv = jnp.dot(h, bd2_ref[...], preferred_element_type=jnp.float32)
  c_ref[...] = cv
  s3_ref[...] = jnp.dot(cv, p3_ref[...],
                        preferred_element_type=jnp.float32) * d8


def _stage_sstep(rp_ref, d8_ref, c_ref, pj_ref, s_ref):
  d8 = d8_ref[...]
  t = (rp_ref[0] + rp_ref[1]) * d8 + jnp.dot(
      c_ref[...], pj_ref[...], preferred_element_type=jnp.float32)
  s_ref[...] = t * d8


def _stage_final(rp_ref, d8_ref, c_ref, p0_ref, psw_ref, b2t_ref, out_ref):
  d8 = d8_ref[...]
  o = (rp_ref[0] + rp_ref[1]) * d8 + jnp.dot(
      c_ref[...], p0_ref[...], preferred_element_type=jnp.float32)
  o = o + b2t_ref[...][None, :]
  osw = jnp.dot(o, psw_ref[...], preferred_element_type=jnp.float32)
  mx = jnp.maximum(o, osw)
  lse = mx + jnp.log(jnp.exp(o - mx) + jnp.exp(osw - mx))
  out_ref[...] = o - lse


def kernel(x, edge_index, W1, b1, W2, b2):
  n0, f = x.shape
  e = edge_index.shape[1]
  nchunks = e // CHUNK
  align = N_SUB * 8 * _GRID
  n = ((n0 + align - 1) // align) * align
  v = n // 16

  xv = jnp.pad(x, ((0, n - n0), (0, 8 - f))).reshape(v, 128)
  w1p = jnp.pad(W1, ((0, 0), (0, 8 - f), (0, 0)))
  row3d = edge_index[0].reshape(nchunks, CHUNK_T, CHUNK_I)
  col3d = edge_index[1].reshape(nchunks, CHUNK_T, CHUNK_I)

  zeros8 = jnp.zeros((n // N_SUB // 2, 8), jnp.float32)
  ones8 = jnp.ones((n, 8), jnp.float32)

  eye16 = jnp.eye(16, dtype=jnp.float32)
  bd1 = jnp.stack([jnp.kron(eye16, w1p[k]) for k in range(4)])
  b1t = jnp.tile(b1, 16)
  bd2 = jnp.kron(eye16, jnp.concatenate(list(W2), axis=1))
  sel = []
  for j in range(4):
    ej = jnp.zeros((8, 8), jnp.float32).at[2 * j, 0].set(1.0).at[
        2 * j + 1, 1].set(1.0)
    sel.append(jnp.kron(eye16, ej))
  esw = jnp.zeros((8, 8), jnp.float32).at[0, 1].set(1.0).at[1, 0].set(1.0)
  psw = jnp.kron(eye16, esw)
  b2t = jnp.tile(jnp.pad(b2, (0, 6)), 16)

  deg_k = _sc_prop(n, e, 8, gather=False)
  prop8 = _sc_prop(n, e, 8, gather=True)

  def as2d(a):
    return a.reshape(n, 8)

  def asv(a):
    return a.reshape(2, v, 128)

  fmat = ('F', (128, 128))
  degp = deg_k(ones8, row3d, col3d, zeros8)
  d8, p0 = _tc_call(_stage_a, v, [2, 1], 2)(asv(degp), xv)

  q1 = prop8(as2d(p0), row3d, col3d, zeros8)
  p1, = _tc_call(_stage_pk, v, [2, 1], 1)(asv(q1), d8)
  q2 = prop8(as2d(p1), row3d, col3d, zeros8)
  p2, = _tc_call(_stage_pk, v, [2, 1], 1)(asv(q2), d8)
  q3 = prop8(as2d(p2), row3d, col3d, zeros8)

  c, s3 = _tc_call(
      _stage_mid, v,
      [1, 2, 2, 2, 1, ('F', (4, 128, 256)), ('F', (256,)),
       ('F', (256, 128)), fmat], 2)(
          xv, asv(q1), asv(q2), asv(q3), d8, bd1, b1t, bd2, sel[3])

  r3 = prop8(as2d(s3), row3d, col3d, zeros8)
  s2, = _tc_call(_stage_sstep, v, [2, 1, 1, fmat], 1)(asv(r3), d8, c, sel[2])
  r2 = prop8(as2d(s2), row3d, col3d, zeros8)
  s1, = _tc_call(_stage_sstep, v, [2, 1, 1, fmat], 1)(asv(r2), d8, c, sel[1])
  r1 = prop8(as2d(s1), row3d, col3d, zeros8)

  outv, = _tc_call(_stage_final, v, [2, 1, 1, fmat, fmat, ('F', (128,))],
                   1)(asv(r1), d8, c, sel[0], psw, b2t)
  return outv.reshape(n, 8)[:n0, :2]


# speedup vs baseline: 1.6092x; 1.6092x over previous
import jax
import jax.numpy as jnp
from jax import lax
from jax.experimental import pallas as pl
from jax.experimental.pallas import tpu as pltpu
from jax.experimental.pallas import tpu_sc as plsc

N_SUB = 16
N_CORE = 2
CHUNK_T = 8
CHUNK_I = 128
CHUNK = CHUNK_T * CHUNK_I
NW = N_CORE * N_SUB


def _sc_prop(n, e, w, gather):
  nchunks = e // CHUNK
  npr = n // N_SUB
  nz = npr // 2

  mesh = plsc.VectorSubcoreMesh(core_axis_name="c", subcore_axis_name="s",
                                num_cores=N_CORE, num_subcores=N_SUB)

  scratch = [
      pltpu.VMEM((2, CHUNK_T, CHUNK_I), jnp.int32),
      pltpu.VMEM((3, CHUNK_T, CHUNK_I), jnp.int32),
      pltpu.VMEM((2, CHUNK, w), jnp.float32),
      pltpu.VMEM_SHARED((n, w), jnp.float32),
      pltpu.SemaphoreType.DMA((2,)),
      pltpu.SemaphoreType.DMA((3,)),
      pltpu.SemaphoreType.DMA,
      pltpu.SemaphoreType.DMA((2,)),
  ]
  if gather:
    scratch.insert(3, pltpu.VMEM_SHARED((n, w), jnp.float32))

  def body(src_hbm, row_hbm, col_hbm, zero_hbm, out_hbm, idxr, idxc, rows,
           *rest):
    if gather:
      src_sp, acc_sp, irsem, icsem, gsem, ssem = rest
    else:
      acc_sp, irsem, icsem, gsem, ssem = rest
      src_sp = None
    c = lax.axis_index("c")
    s = lax.axis_index("s")
    wid = c * N_SUB + s
    base = s * npr

    if gather:
      pltpu.sync_copy(src_hbm.at[pl.ds(base, npr)],
                      src_sp.at[pl.ds(base, npr)])
    else:
      pltpu.sync_copy(src_hbm.at[pl.ds(0, CHUNK)], rows.at[0])
    for z in range(2):
      pltpu.sync_copy(zero_hbm, acc_sp.at[pl.ds(base + z * nz, nz)])
    plsc.subcore_barrier()

    ntrips = (nchunks - wid + NW - 1) // NW

    def start_idx(i):
      cid = wid + i * NW
      if gather:
        pltpu.async_copy(row_hbm.at[cid], idxr.at[i % 2], irsem.at[i % 2])
      pltpu.async_copy(col_hbm.at[cid], idxc.at[i % 3], icsem.at[i % 3])

    def wait_scatters(p):
      for j in range(CHUNK_T):
        pltpu.make_async_copy(rows.at[0, pl.ds(j * CHUNK_I, CHUNK_I)],
                              acc_sp.at[idxc.at[0, 0]], ssem.at[p]).wait()

    start_idx(0)

    def trip(i, carry):
      b2 = i % 2
      b3 = i % 3

      @pl.when(i >= 2)
      def _():
        wait_scatters(b2)

      @pl.when(i + 1 < ntrips)
      def _():
        start_idx(i + 1)

      if gather:
        pltpu.make_async_copy(row_hbm.at[0], idxr.at[b2],
                              irsem.at[b2]).wait()
      pltpu.make_async_copy(col_hbm.at[0], idxc.at[b3], icsem.at[b3]).wait()

      if gather:
        gcps = [
            pltpu.async_copy(src_sp.at[idxr.at[b2, j]],
                             rows.at[b2, pl.ds(j * CHUNK_I, CHUNK_I)], gsem)
            for j in range(CHUNK_T)
        ]
        for cp in gcps:
          cp.wait()
      for j in range(CHUNK_T):
        pltpu.async_copy(
            rows.at[b2 if gather else 0, pl.ds(j * CHUNK_I, CHUNK_I)],
            acc_sp.at[idxc.at[b3, j]], ssem.at[b2], add=True)
      return carry

    lax.fori_loop(0, ntrips, trip, 0)
    wait_scatters(0)
    wait_scatters(1)
    plsc.subcore_barrier()

    pltpu.sync_copy(acc_sp.at[pl.ds(base, npr)],
                    out_hbm.at[c, pl.ds(base, npr)])

  return pl.kernel(
      body,
      out_type=jax.ShapeDtypeStruct((N_CORE, n, w), jnp.float32),
      mesh=mesh,
      scratch_types=scratch,
      compiler_params=pltpu.CompilerParams(use_tc_tiling_on_sc=False),
  )


_GRID = 8


def _tc_call(fn, v, in_kinds, n_out):
  rv = v // _GRID

  def spec(kd):
    if isinstance(kd, tuple) and kd[0] == 'F':
      shape = kd[1]
      return pl.BlockSpec(shape, lambda i: (0,) * len(shape))
    if kd == 2:
      return pl.BlockSpec((2, rv, 128), lambda i: (0, i, 0))
    return pl.BlockSpec((rv, 128), lambda i: (i, 0))

  return pl.pallas_call(
      fn,
      grid=(_GRID,),
      in_specs=[spec(kd) for kd in in_kinds],
      out_specs=[spec(1) for _ in range(n_out)],
      out_shape=[jax.ShapeDtypeStruct((v, 128), jnp.float32)] * n_out,
  )


def _stage_a(degp_ref, x_ref, d8_ref, p0_ref):
  deg = degp_ref[0] + degp_ref[1]
  d8 = jnp.where(deg > 0, lax.rsqrt(jnp.maximum(deg, 1e-12)), 0.0)
  d8_ref[...] = d8
  p0_ref[...] = x_ref[...] * d8


def _stage_pk(qp_ref, d8_ref, pk_ref):
  d8 = d8_ref[...]
  pk_ref[...] = (qp_ref[0] + qp_ref[1]) * d8 * d8


def _stage_mid(x_ref, q1_ref, q2_ref, q3_ref, d8_ref, bd1_ref, b1t_ref,
               bd2_ref, p3_ref, c_ref, s3_ref):
  d8 = d8_ref[...]
  g1 = (q1_ref[0] + q1_ref[1]) * d8
  g2 = (q2_ref[0] + q2_ref[1]) * d8
  g3 = (q3_ref[0] + q3_ref[1]) * d8
  o = jnp.dot(x_ref[...], bd1_ref[0], preferred_element_type=jnp.float32)
  o += jnp.dot(g1, bd1_ref[1], preferred_element_type=jnp.float32)
  o += jnp.dot(g2, bd1_ref[2], preferred_element_type=jnp.float32)
  o += jnp.dot(g3, bd1_ref[3], preferred_element_type=jnp.float32)
  h = jnp.maximum(o + b1t_ref[...][None, :], 0.0)
  cv = jnp.dot(h, bd2_ref[...], preferred_element_type=jnp.float32)
  c_ref[...] = cv
  s3_ref[...] = jnp.dot(cv, p3_ref[...],
                        preferred_element_type=jnp.float32) * d8


def _stage_sstep(rp_ref, d8_ref, c_ref, pj_ref, s_ref):
  d8 = d8_ref[...]
  t = (rp_ref[0] + rp_ref[1]) * d8 + jnp.dot(
      c_ref[...], pj_ref[...], preferred_element_type=jnp.float32)
  s_ref[...] = t * d8


def _stage_final(rp_ref, d8_ref, c_ref, p0_ref, psw_ref, b2t_ref, cmp_ref,
                 out_ref):
  d8 = d8_ref[...]
  o = (rp_ref[0] + rp_ref[1]) * d8 + jnp.dot(
      c_ref[...], p0_ref[...], preferred_element_type=jnp.float32)
  o = o + b2t_ref[...][None, :]
  osw = jnp.dot(o, psw_ref[...], preferred_element_type=jnp.float32)
  mx = jnp.maximum(o, osw)
  lse = mx + jnp.log(jnp.exp(o - mx) + jnp.exp(osw - mx))
  res = o - lse
  out_ref[...] = jnp.dot(res, cmp_ref[...], preferred_element_type=jnp.float32)


def kernel(x, edge_index, W1, b1, W2, b2):
  n0, f = x.shape
  e = edge_index.shape[1]
  nchunks = e // CHUNK
  align = N_SUB * 8 * _GRID
  n = ((n0 + align - 1) // align) * align
  v = n // 16

  xv = jnp.pad(x, ((0, n - n0), (0, 8 - f))).reshape(v, 128)
  w1p = jnp.pad(W1, ((0, 0), (0, 8 - f), (0, 0)))
  row3d = edge_index[0].reshape(nchunks, CHUNK_T, CHUNK_I)
  col3d = edge_index[1].reshape(nchunks, CHUNK_T, CHUNK_I)

  zeros8 = jnp.zeros((n // N_SUB // 2, 8), jnp.float32)
  ones8 = jnp.ones((n, 8), jnp.float32)

  eye16 = jnp.eye(16, dtype=jnp.float32)
  bd1 = jnp.stack([jnp.kron(eye16, w1p[k]) for k in range(4)])
  b1t = jnp.tile(b1, 16)
  bd2 = jnp.kron(eye16, jnp.concatenate(list(W2), axis=1))
  sel = []
  for j in range(4):
    ej = jnp.zeros((8, 8), jnp.float32).at[2 * j, 0].set(1.0).at[
        2 * j + 1, 1].set(1.0)
    sel.append(jnp.kron(eye16, ej))
  esw = jnp.zeros((8, 8), jnp.float32).at[0, 1].set(1.0).at[1, 0].set(1.0)
  psw = jnp.kron(eye16, esw)
  b2t = jnp.tile(jnp.pad(b2, (0, 6)), 16)

  deg_k = _sc_prop(n, e, 8, gather=False)
  prop8 = _sc_prop(n, e, 8, gather=True)

  def as2d(a):
    return a.reshape(n, 8)

  def asv(a):
    return a.reshape(2, v, 128)

  fmat = ('F', (128, 128))
  degp = deg_k(ones8, row3d, col3d, zeros8)
  d8, p0 = _tc_call(_stage_a, v, [2, 1], 2)(asv(degp), xv)

  q1 = prop8(as2d(p0), row3d, col3d, zeros8)
  p1, = _tc_call(_stage_pk, v, [2, 1], 1)(asv(q1), d8)
  q2 = prop8(as2d(p1), row3d, col3d, zeros8)
  p2, = _tc_call(_stage_pk, v, [2, 1], 1)(asv(q2), d8)
  q3 = prop8(as2d(p2), row3d, col3d, zeros8)

  c, s3 = _tc_call(
      _stage_mid, v,
      [1, 2, 2, 2, 1, ('F', (4, 128, 256)), ('F', (256,)),
       ('F', (256, 128)), fmat], 2)(
          xv, asv(q1), asv(q2), asv(q3), d8, bd1, b1t, bd2, sel[3])

  r3 = prop8(as2d(s3), row3d, col3d, zeros8)
  s2, = _tc_call(_stage_sstep, v, [2, 1, 1, fmat], 1)(asv(r3), d8, c, sel[2])
  r2 = prop8(as2d(s2), row3d, col3d, zeros8)
  s1, = _tc_call(_stage_sstep, v, [2, 1, 1, fmat], 1)(asv(r2), d8, c, sel[1])
  r1 = prop8(as2d(s1), row3d, col3d, zeros8)

  cmp = jnp.zeros((128, 32), jnp.float32)
  for m in range(16):
    cmp = cmp.at[8 * m, 2 * m].set(1.0).at[8 * m + 1, 2 * m + 1].set(1.0)
  rv = v // _GRID
  outp = pl.pallas_call(
      _stage_final,
      grid=(_GRID,),
      in_specs=[pl.BlockSpec((2, rv, 128), lambda i: (0, i, 0)),
                pl.BlockSpec((rv, 128), lambda i: (i, 0)),
                pl.BlockSpec((rv, 128), lambda i: (i, 0)),
                pl.BlockSpec((128, 128), lambda i: (0, 0)),
                pl.BlockSpec((128, 128), lambda i: (0, 0)),
                pl.BlockSpec((128,), lambda i: (0,)),
                pl.BlockSpec((128, 32), lambda i: (0, 0))],
      out_specs=pl.BlockSpec((rv, 32), lambda i: (i, 0)),
      out_shape=jax.ShapeDtypeStruct((v, 32), jnp.float32),
  )(asv(r1), d8, c, sel[0], psw, b2t, cmp)
  return outp.reshape(n, 2)[:n0]


# speedup vs baseline: 1.6237x; 1.0091x over previous
import jax
import jax.numpy as jnp
from jax import lax
from jax.experimental import pallas as pl
from jax.experimental.pallas import tpu as pltpu
from jax.experimental.pallas import tpu_sc as plsc

N_SUB = 16
N_CORE = 2
CHUNK_T = 10
CHUNK_I = 128
CHUNK = CHUNK_T * CHUNK_I
NW = N_CORE * N_SUB


def _sc_prop(n, e, w, gather):
  nchunks = e // CHUNK
  npr = n // N_SUB
  nz = npr // 2

  mesh = plsc.VectorSubcoreMesh(core_axis_name="c", subcore_axis_name="s",
                                num_cores=N_CORE, num_subcores=N_SUB)

  scratch = [
      pltpu.VMEM((2, CHUNK_T, CHUNK_I), jnp.int32),
      pltpu.VMEM((3, CHUNK_T, CHUNK_I), jnp.int32),
      pltpu.VMEM((2, CHUNK, w), jnp.float32),
      pltpu.VMEM_SHARED((n, w), jnp.float32),
      pltpu.SemaphoreType.DMA((2,)),
      pltpu.SemaphoreType.DMA((3,)),
      pltpu.SemaphoreType.DMA,
      pltpu.SemaphoreType.DMA((2,)),
  ]
  if gather:
    scratch.insert(3, pltpu.VMEM_SHARED((n, w), jnp.float32))

  def body(src_hbm, row_hbm, col_hbm, zero_hbm, out_hbm, idxr, idxc, rows,
           *rest):
    if gather:
      src_sp, acc_sp, irsem, icsem, gsem, ssem = rest
    else:
      acc_sp, irsem, icsem, gsem, ssem = rest
      src_sp = None
    c = lax.axis_index("c")
    s = lax.axis_index("s")
    wid = c * N_SUB + s
    base = s * npr

    if gather:
      pltpu.sync_copy(src_hbm.at[pl.ds(base, npr)],
                      src_sp.at[pl.ds(base, npr)])
    else:
      pltpu.sync_copy(src_hbm.at[pl.ds(0, CHUNK)], rows.at[0])
    for z in range(2):
      pltpu.sync_copy(zero_hbm, acc_sp.at[pl.ds(base + z * nz, nz)])
    plsc.subcore_barrier()

    ntrips = (nchunks - wid + NW - 1) // NW

    def start_idx(i):
      cid = wid + i * NW
      if gather:
        pltpu.async_copy(row_hbm.at[cid], idxr.at[i % 2], irsem.at[i % 2])
      pltpu.async_copy(col_hbm.at[cid], idxc.at[i % 3], icsem.at[i % 3])

    def wait_scatters(p):
      for j in range(CHUNK_T):
        pltpu.make_async_copy(rows.at[0, pl.ds(j * CHUNK_I, CHUNK_I)],
                              acc_sp.at[idxc.at[0, 0]], ssem.at[p]).wait()

    start_idx(0)

    def trip(i, carry):
      b2 = i % 2
      b3 = i % 3

      @pl.when(i >= 2)
      def _():
        wait_scatters(b2)

      @pl.when(i + 1 < ntrips)
      def _():
        start_idx(i + 1)

      if gather:
        pltpu.make_async_copy(row_hbm.at[0], idxr.at[b2],
                              irsem.at[b2]).wait()
      pltpu.make_async_copy(col_hbm.at[0], idxc.at[b3], icsem.at[b3]).wait()

      if gather:
        gcps = [
            pltpu.async_copy(src_sp.at[idxr.at[b2, j]],
                             rows.at[b2, pl.ds(j * CHUNK_I, CHUNK_I)], gsem)
            for j in range(CHUNK_T)
        ]
        for cp in gcps:
          cp.wait()
      for j in range(CHUNK_T):
        pltpu.async_copy(
            rows.at[b2 if gather else 0, pl.ds(j * CHUNK_I, CHUNK_I)],
            acc_sp.at[idxc.at[b3, j]], ssem.at[b2], add=True)
      return carry

    lax.fori_loop(0, ntrips, trip, 0)
    wait_scatters(0)
    wait_scatters(1)
    plsc.subcore_barrier()

    pltpu.sync_copy(acc_sp.at[pl.ds(base, npr)],
                    out_hbm.at[c, pl.ds(base, npr)])

  return pl.kernel(
      body,
      out_type=jax.ShapeDtypeStruct((N_CORE, n, w), jnp.float32),
      mesh=mesh,
      scratch_types=scratch,
      compiler_params=pltpu.CompilerParams(use_tc_tiling_on_sc=False),
  )


_GRID = 8


def _tc_call(fn, v, in_kinds, n_out):
  rv = v // _GRID

  def spec(kd):
    if isinstance(kd, tuple) and kd[0] == 'F':
      shape = kd[1]
      return pl.BlockSpec(shape, lambda i: (0,) * len(shape))
    if kd == 2:
      return pl.BlockSpec((2, rv, 128), lambda i: (0, i, 0))
    return pl.BlockSpec((rv, 128), lambda i: (i, 0))

  return pl.pallas_call(
      fn,
      grid=(_GRID,),
      in_specs=[spec(kd) for kd in in_kinds],
      out_specs=[spec(1) for _ in range(n_out)],
      out_shape=[jax.ShapeDtypeStruct((v, 128), jnp.float32)] * n_out,
  )


def _stage_a(degp_ref, x_ref, d8_ref, p0_ref):
  deg = degp_ref[0] + degp_ref[1]
  d8 = jnp.where(deg > 0, lax.rsqrt(jnp.maximum(deg, 1e-12)), 0.0)
  d8_ref[...] = d8
  p0_ref[...] = x_ref[...] * d8


def _stage_pk(qp_ref, d8_ref, pk_ref):
  d8 = d8_ref[...]
  pk_ref[...] = (qp_ref[0] + qp_ref[1]) * d8 * d8


def _stage_mid(x_ref, q1_ref, q2_ref, q3_ref, d8_ref, bd1_ref, b1t_ref,
               bd2_ref, p3_ref, c_ref, s3_ref):
  d8 = d8_ref[...]
  g1 = (q1_ref[0] + q1_ref[1]) * d8
  g2 = (q2_ref[0] + q2_ref[1]) * d8
  g3 = (q3_ref[0] + q3_ref[1]) * d8
  o = jnp.dot(x_ref[...], bd1_ref[0], preferred_element_type=jnp.float32)
  o += jnp.dot(g1, bd1_ref[1], preferred_element_type=jnp.float32)
  o += jnp.dot(g2, bd1_ref[2], preferred_element_type=jnp.float32)
  o += jnp.dot(g3, bd1_ref[3], preferred_element_type=jnp.float32)
  h = jnp.maximum(o + b1t_ref[...][None, :], 0.0)
  cv = jnp.dot(h, bd2_ref[...], preferred_element_type=jnp.float32)
  c_ref[...] = cv
  s3_ref[...] = jnp.dot(cv, p3_ref[...],
                        preferred_element_type=jnp.float32) * d8


def _stage_sstep(rp_ref, d8_ref, c_ref, pj_ref, s_ref):
  d8 = d8_ref[...]
  t = (rp_ref[0] + rp_ref[1]) * d8 + jnp.dot(
      c_ref[...], pj_ref[...], preferred_element_type=jnp.float32)
  s_ref[...] = t * d8


def _stage_final(rp_ref, d8_ref, c_ref, p0_ref, psw_ref, b2t_ref, cmp_ref,
                 out_ref):
  d8 = d8_ref[...]
  o = (rp_ref[0] + rp_ref[1]) * d8 + jnp.dot(
      c_ref[...], p0_ref[...], preferred_element_type=jnp.float32)
  o = o + b2t_ref[...][None, :]
  osw = jnp.dot(o, psw_ref[...], preferred_element_type=jnp.float32)
  mx = jnp.maximum(o, osw)
  lse = mx + jnp.log(jnp.exp(o - mx) + jnp.exp(osw - mx))
  res = o - lse
  out_ref[...] = jnp.dot(res, cmp_ref[...], preferred_element_type=jnp.float32)


def kernel(x, edge_index, W1, b1, W2, b2):
  n0, f = x.shape
  e = edge_index.shape[1]
  nchunks = e // CHUNK
  align = N_SUB * 8 * _GRID
  n = ((n0 + align - 1) // align) * align
  v = n // 16

  xv = jnp.pad(x, ((0, n - n0), (0, 8 - f))).reshape(v, 128)
  w1p = jnp.pad(W1, ((0, 0), (0, 8 - f), (0, 0)))
  row3d = edge_index[0].reshape(nchunks, CHUNK_T, CHUNK_I)
  col3d = edge_index[1].reshape(nchunks, CHUNK_T, CHUNK_I)

  zeros8 = jnp.zeros((n // N_SUB // 2, 8), jnp.float32)
  ones8 = jnp.ones((n, 8), jnp.float32)

  eye16 = jnp.eye(16, dtype=jnp.float32)
  bd1 = jnp.stack([jnp.kron(eye16, w1p[k]) for k in range(4)])
  b1t = jnp.tile(b1, 16)
  bd2 = jnp.kron(eye16, jnp.concatenate(list(W2), axis=1))
  sel = []
  for j in range(4):
    ej = jnp.zeros((8, 8), jnp.float32).at[2 * j, 0].set(1.0).at[
        2 * j + 1, 1].set(1.0)
    sel.append(jnp.kron(eye16, ej))
  esw = jnp.zeros((8, 8), jnp.float32).at[0, 1].set(1.0).at[1, 0].set(1.0)
  psw = jnp.kron(eye16, esw)
  b2t = jnp.tile(jnp.pad(b2, (0, 6)), 16)

  deg_k = _sc_prop(n, e, 8, gather=False)
  prop8 = _sc_prop(n, e, 8, gather=True)

  def as2d(a):
    return a.reshape(n, 8)

  def asv(a):
    return a.reshape(2, v, 128)

  fmat = ('F', (128, 128))
  degp = deg_k(ones8, row3d, col3d, zeros8)
  d8, p0 = _tc_call(_stage_a, v, [2, 1], 2)(asv(degp), xv)

  q1 = prop8(as2d(p0), row3d, col3d, zeros8)
  p1, = _tc_call(_stage_pk, v, [2, 1], 1)(asv(q1), d8)
  q2 = prop8(as2d(p1), row3d, col3d, zeros8)
  p2, = _tc_call(_stage_pk, v, [2, 1], 1)(asv(q2), d8)
  q3 = prop8(as2d(p2), row3d, col3d, zeros8)

  c, s3 = _tc_call(
      _stage_mid, v,
      [1, 2, 2, 2, 1, ('F', (4, 128, 256)), ('F', (256,)),
       ('F', (256, 128)), fmat], 2)(
          xv, asv(q1), asv(q2), asv(q3), d8, bd1, b1t, bd2, sel[3])

  r3 = prop8(as2d(s3), row3d, col3d, zeros8)
  s2, = _tc_call(_stage_sstep, v, [2, 1, 1, fmat], 1)(asv(r3), d8, c, sel[2])
  r2 = prop8(as2d(s2), row3d, col3d, zeros8)
  s1, = _tc_call(_stage_sstep, v, [2, 1, 1, fmat], 1)(asv(r2), d8, c, sel[1])
  r1 = prop8(as2d(s1), row3d, col3d, zeros8)

  cmp = jnp.zeros((128, 32), jnp.float32)
  for m in range(16):
    cmp = cmp.at[8 * m, 2 * m].set(1.0).at[8 * m + 1, 2 * m + 1].set(1.0)
  rv = v // _GRID
  outp = pl.pallas_call(
      _stage_final,
      grid=(_GRID,),
      in_specs=[pl.BlockSpec((2, rv, 128), lambda i: (0, i, 0)),
                pl.BlockSpec((rv, 128), lambda i: (i, 0)),
                pl.BlockSpec((rv, 128), lambda i: (i, 0)),
                pl.BlockSpec((128, 128), lambda i: (0, 0)),
                pl.BlockSpec((128, 128), lambda i: (0, 0)),
                pl.BlockSpec((128,), lambda i: (0,)),
                pl.BlockSpec((128, 32), lambda i: (0, 0))],
      out_specs=pl.BlockSpec((rv, 32), lambda i: (i, 0)),
      out_shape=jax.ShapeDtypeStruct((v, 32), jnp.float32),
  )(asv(r1), d8, c, sel[0], psw, b2t, cmp)
  return outp.reshape(n, 2)[:n0]
